# baseline (speedup 1.0000x reference)
import jax
import jax.numpy as jnp
import numpy as np
from jax import lax
from jax.experimental import pallas as pl
from jax.experimental.pallas import tpu as pltpu
from jax.experimental.pallas import tpu_sc as plsc

_A_IDX = [0, 0, 0, 0, 0, 0, 1, 1, 1, 1, 1, 2, 2, 2, 2, 1, 1, 1, 0, 0, 0]
_B_IDX = [0, 0, 0, 0, 0, 0, 1, 1, 1, 1, 1, 2, 2, 2, 2, 2, 2, 2, 2, 2, 1]
_C_IDX = [0, 1, 2, 1, 0, 0, 1, 2, 1, 0, 0, 2, 1, 0, 0, 1, 0, 0, 0, 0, 0]
_D_IDX = [0, 1, 2, 2, 2, 1, 1, 2, 2, 2, 1, 2, 2, 2, 1, 2, 2, 1, 2, 1, 1]


def _tables():
    flat = [27 * a + 9 * b + 3 * c + d
            for a, b, c, d in zip(_A_IDX, _B_IDX, _C_IDX, _D_IDX)]
    rows, cols = np.triu_indices(6)
    s2 = np.sqrt(2)
    m = np.array([[1, 1, 1, s2, s2, s2],
                  [1, 1, 1, s2, s2, s2],
                  [1, 1, 1, s2, s2, s2],
                  [s2, s2, s2, 2, 2, 2],
                  [s2, s2, s2, 2, 2, 2],
                  [s2, s2, s2, 2, 2, 2]], dtype=np.float32)
    comp_of = {}
    for k, (r, c) in enumerate(zip(rows, cols)):
        comp_of[(r, c)] = k
        comp_of[(c, r)] = k
    scale = [float(m[r, c]) for r, c in zip(rows, cols)]
    out_comp = [comp_of[(i, j)] for i in range(6) for j in range(6)]
    return flat, scale, out_comp, m

_FLAT, _SCALE, _OUT_COMP, _MASK = _tables()
_NK = 21
_SLABS = [(0, 0, 0, 3), (0, 1, 0, 1), (0, 2, 0, 1),
          (1, 1, 0, 3), (1, 2, 0, 2), (2, 2, 0, 3)]
_NG = len(_SLABS)


def _slab_of(f):
    ab, c, d = f // 9, (f // 3) % 3, f % 3
    a, b = ab // 3, ab % 3
    for s, (sa, sb, c0, nc) in enumerate(_SLABS):
        if sa == a and sb == b and c0 <= c < c0 + nc:
            return s, c - c0, d
    raise AssertionError(f)

_GRP = [_slab_of(f) for f in _FLAT]

_NB = 500000
_W = 384
_NCHUNK = _NB // _W
_TAIL = _NB - _NCHUNK * _W
_NW = 32
_ITERS = -(-_NCHUNK // _NW)
_ITERS_2 = -(-_ITERS // 2)


def _body(c_hbm, o_hbm, *rest):
    ins = (rest[:_NG], rest[_NG:2 * _NG])
    base = 2 * _NG
    outs = (rest[base], rest[base + 1])
    isems = rest[base + 2:base + 4]
    osems = rest[base + 4:base + 6]

    wid = lax.axis_index("s") * 2 + lax.axis_index("c")

    def in_copies(m, slot):
        n0 = (wid + m * _NW) * _W
        cps = []
        for g, (a, b, c0, nc) in enumerate(_SLABS):
            cps.append(pltpu.make_async_copy(
                c_hbm.at[a, b, pl.ds(c0, nc), :, pl.ds(n0, _W)],
                ins[slot][g], isems[slot]))
        return cps

    def out_copies(m, slot):
        n0 = (wid + m * _NW) * _W
        return [pltpu.make_async_copy(
                    outs[slot], o_hbm.at[:, :, pl.ds(n0, _W)],
                    osems[slot])]

    for cp in in_copies(0, 0):
        cp.start()

    def iter_body(it, _):
        for b in range(2):
            m = 2 * it + b
            chunk = wid + m * _NW
            valid = chunk < _NCHUNK

            @pl.when(valid)
            def _():
                for cp in in_copies(m, b):
                    cp.wait()

            @pl.when(wid + (m + 1) * _NW < _NCHUNK)
            def _():
                for cp in in_copies(m + 1, 1 - b):
                    cp.start()

            @pl.when(valid & (m >= 2))
            def _():
                for cp in out_copies(m - 2, b):
                    cp.wait()

            @pl.when(valid)
            def _():
                @plsc.parallel_loop(0, _W // 16, 1, unroll=2)
                def rep_step(g):
                    o = g * 16
                    vals = []
                    for k in range(_NK):
                        s, c, d = _GRP[k]
                        v = ins[b][s][c, d, pl.ds(o, 16)]
                        if _SCALE[k] != 1.0:
                            v = v * _SCALE[k]
                        vals.append(v)
                    for j36 in range(36):
                        i, j = divmod(j36, 6)
                        outs[b][i, j, pl.ds(o, 16)] = vals[_OUT_COMP[j36]]

                for cp in out_copies(m, b):
                    cp.start()

        return 0

    lax.fori_loop(0, _ITERS_2, iter_body, 0)

    for m in (2 * _ITERS_2 - 2, 2 * _ITERS_2 - 1):
        chunk = wid + m * _NW

        @pl.when(chunk < _NCHUNK)
        def _():
            for cp in out_copies(m, m % 2):
                cp.wait()


@jax.jit
def kernel(C):
    c5 = jnp.transpose(C, (1, 2, 3, 4, 0))
    mesh = plsc.VectorSubcoreMesh(core_axis_name="c", subcore_axis_name="s")
    scratch = [pltpu.VMEM((nc, 3, _W), jnp.float32)
               for _ in range(2) for (_, _, _, nc) in _SLABS]
    scratch += [pltpu.VMEM((6, 6, _W), jnp.float32) for _ in range(2)]
    scratch += [pltpu.SemaphoreType.DMA] * 4
    o_t = pl.kernel(
        _body,
        out_type=jax.ShapeDtypeStruct((6, 6, _NB), jnp.float32),
        mesh=mesh,
        scratch_types=scratch,
        compiler_params=pltpu.CompilerParams(needs_layout_passes=False),
    )(c5)
    ta = jnp.asarray([_A_IDX[k] for k in _OUT_COMP])
    tb = jnp.asarray([_B_IDX[k] for k in _OUT_COMP])
    tc = jnp.asarray([_C_IDX[k] for k in _OUT_COMP])
    td = jnp.asarray([_D_IDX[k] for k in _OUT_COMP])
    tail = C[_NCHUNK * _W:]
    tv = tail[:, ta, tb, tc, td] * jnp.asarray(_MASK.reshape(36))
    o_t = lax.dynamic_update_slice(
        o_t, jnp.transpose(tv, (1, 0)).reshape(6, 6, _TAIL),
        (0, 0, _NCHUNK * _W))
    return jnp.transpose(o_t, (2, 0, 1))
